# baseline (speedup 1.0000x reference)
import functools

import jax
import jax.numpy as jnp
from jax import lax
from jax.experimental import pallas as pl
from jax.experimental.pallas import tpu as pltpu
from jax.experimental.pallas import tpu_sc as plsc

N_ITEMS = 1000000
EMBED_DIM = 64
BATCH = 16384

NC = 2
NS = 16
NW = NC * NS
B_PER_W = BATCH // NW
LANES = 16
W = 128
NWIN = B_PER_W // W

REPACK_C = 7936
REPACK_Q = REPACK_C // 4
N_STEPS = -(-N_ITEMS // REPACK_C)
T2_ROWS = N_STEPS * REPACK_Q
HALF = EMBED_DIM // 2


def _repack_body(tt_ref, out_ref):
    x = tt_ref[...]
    u = lax.bitcast_convert_type(x, jnp.int32)
    lo = lax.shift_right_logical(u[:HALF, :], 16)
    hi = u[HALF:, :]
    w = lax.bitwise_or(lo, lax.bitwise_and(hi, jnp.int32(-65536)))
    for q in range(4):
        out_ref[:, q * HALF:(q + 1) * HALF] = (
            w[:, q * REPACK_Q:(q + 1) * REPACK_Q].T)


def _tc_repack(table_t):
    return pl.pallas_call(
        _repack_body,
        grid=(N_STEPS,),
        in_specs=[pl.BlockSpec((EMBED_DIM, REPACK_C), lambda i: (0, i))],
        out_specs=pl.BlockSpec((REPACK_Q, 128), lambda i: (i, 0)),
        out_shape=jax.ShapeDtypeStruct((T2_ROWS, 128), jnp.int32),
    )(table_t)


def _sc_gather(gid2, po, table2):
    mesh = plsc.VectorSubcoreMesh(core_axis_name="c", subcore_axis_name="s")

    @functools.partial(
        pl.kernel,
        mesh=mesh,
        out_type=jax.ShapeDtypeStruct((BATCH, HALF), jnp.int32),
        scratch_types=[
            pltpu.VMEM((NWIN, W), jnp.int32),
            pltpu.VMEM((B_PER_W,), jnp.int32),
            pltpu.VMEM((NWIN, W, 128), jnp.int32),
            pltpu.VMEM((W, HALF), jnp.int32),
            [pltpu.SemaphoreType.DMA] * NWIN,
        ],
    )
    def gather_k(gid_hbm, po_hbm, table_hbm, out_hbm,
                 gid_v, po_v, bufs, stage, sems):
        wid = lax.axis_index("s") * NC + lax.axis_index("c")
        base = wid * B_PER_W
        pltpu.sync_copy(gid_hbm.at[pl.ds(wid * NWIN, NWIN)], gid_v)
        pltpu.sync_copy(po_hbm.at[pl.ds(base, B_PER_W)], po_v)

        copies = [
            pltpu.async_copy(table_hbm.at[gid_v.at[w]], bufs.at[w], sems[w])
            for w in range(NWIN)
        ]
        for w in range(NWIN):
            copies[w].wait()

            def ext(blk, _, w=w):
                pov = po_v[pl.ds(w * W + blk * LANES, LANES)]
                for l in range(LANES):
                    po_j = pov[l]
                    for k in range(HALF // LANES):
                        stage[blk * LANES + l, pl.ds(k * LANES, LANES)] = (
                            bufs[w, blk * LANES + l,
                                 pl.ds(po_j + k * LANES, LANES)])
                return 0

            lax.fori_loop(0, W // LANES, ext, 0, unroll=False)
            out_base = pl.multiple_of(base + w * W, W)
            pltpu.sync_copy(stage, out_hbm.at[pl.ds(out_base, W)])

    return gather_k(gid2, po, table2)


def _tc_body(embp_ref, feat_ref, w1_ref, b1_ref, w2_ref, b2_ref,
             wc_ref, bc_ref, out_ref):
    dn = (((1,), (1,)), ((), ()))
    u = embp_ref[...]
    lo = lax.bitcast_convert_type(lax.shift_left(u, 16), jnp.float32)
    hi = lax.bitcast_convert_type(
        lax.bitwise_and(u, jnp.int32(-65536)), jnp.float32)
    emb = jnp.concatenate([lo, hi], axis=1)
    f = feat_ref[...]
    h = jnp.maximum(
        lax.dot_general(f, w1_ref[...], dn,
                        preferred_element_type=jnp.float32)
        + b1_ref[...][jnp.newaxis, :], 0.0)
    f2 = (lax.dot_general(h, w2_ref[...], dn,
                          preferred_element_type=jnp.float32)
          + b2_ref[...][jnp.newaxis, :])
    wc = wc_ref[...]
    o = (lax.dot_general(emb, wc[:, :EMBED_DIM], dn,
                         preferred_element_type=jnp.float32)
         + lax.dot_general(f2, wc[:, EMBED_DIM:], dn,
                           preferred_element_type=jnp.float32)
         + bc_ref[...][jnp.newaxis, :])
    s = jnp.sum(o * o, axis=1, keepdims=True)
    out_ref[...] = o * lax.rsqrt(jnp.maximum(s, 1e-24))


def _tc_dense(embp, feats, W1, b1, W2, b2, Wc, bc):
    tb = 2048
    grid = BATCH // tb
    full = lambda shape: pl.BlockSpec(shape, lambda i: tuple([0] * len(shape)))
    return pl.pallas_call(
        _tc_body,
        grid=(grid,),
        in_specs=[
            pl.BlockSpec((tb, HALF), lambda i: (i, 0)),
            pl.BlockSpec((tb, 4), lambda i: (i, 0)),
            full((32, 4)),
            full((32,)),
            full((EMBED_DIM, 32)),
            full((EMBED_DIM,)),
            full((EMBED_DIM, 2 * EMBED_DIM)),
            full((EMBED_DIM,)),
        ],
        out_specs=pl.BlockSpec((tb, EMBED_DIM), lambda i: (i, 0)),
        out_shape=jax.ShapeDtypeStruct((BATCH, EMBED_DIM), jnp.float32),
    )(embp, feats, W1, b1, W2, b2, Wc, bc)


def kernel(item_ids, item_features, emb_table, W1, b1, W2, b2, Wc, bc):
    ids = item_ids.astype(jnp.int32)
    q = ids // REPACK_C
    jj = ids - q * REPACK_C
    h = jj // REPACK_Q
    jr = jj - h * REPACK_Q
    gid2 = (q * REPACK_Q + jr).reshape(BATCH // W, W)
    po = h * HALF

    table2 = _tc_repack(emb_table.T)
    embp = _sc_gather(gid2, po, table2)
    return _tc_dense(embp, item_features, W1, b1, W2, b2, Wc, bc)


# speedup vs baseline: 1.1054x; 1.1054x over previous
import functools

import jax
import jax.numpy as jnp
from jax import lax
from jax.experimental import pallas as pl
from jax.experimental.pallas import tpu as pltpu
from jax.experimental.pallas import tpu_sc as plsc

N_ITEMS = 1000000
EMBED_DIM = 64
BATCH = 16384

NC = 2
NS = 16
NW = NC * NS
B_PER_W = BATCH // NW
LANES = 16
W = 128
NWIN = B_PER_W // W

REPACK_C = 15872
REPACK_Q = REPACK_C // 4
N_STEPS = -(-N_ITEMS // REPACK_C)
T2_ROWS = N_STEPS * REPACK_Q
HALF = EMBED_DIM // 2


def _repack_body(tt_ref, out_ref):
    x = tt_ref[...]
    u = lax.bitcast_convert_type(x, jnp.int32)
    lo = lax.shift_right_logical(u[:HALF, :], 16)
    hi = u[HALF:, :]
    w = lax.bitwise_or(lo, lax.bitwise_and(hi, jnp.int32(-65536)))
    for q in range(4):
        out_ref[:, q * HALF:(q + 1) * HALF] = (
            w[:, q * REPACK_Q:(q + 1) * REPACK_Q].T)


def _tc_repack(table_t):
    return pl.pallas_call(
        _repack_body,
        grid=(N_STEPS,),
        in_specs=[pl.BlockSpec((EMBED_DIM, REPACK_C), lambda i: (0, i))],
        out_specs=pl.BlockSpec((REPACK_Q, 128), lambda i: (i, 0)),
        out_shape=jax.ShapeDtypeStruct((T2_ROWS, 128), jnp.int32),
    )(table_t)


def _sc_gather(gid2, po, table2):
    mesh = plsc.VectorSubcoreMesh(core_axis_name="c", subcore_axis_name="s")

    @functools.partial(
        pl.kernel,
        mesh=mesh,
        out_type=jax.ShapeDtypeStruct((BATCH, HALF), jnp.int32),
        scratch_types=[
            pltpu.VMEM((NWIN, W), jnp.int32),
            pltpu.VMEM((B_PER_W,), jnp.int32),
            pltpu.VMEM((NWIN, W, 128), jnp.int32),
            pltpu.VMEM((W, HALF), jnp.int32),
            [pltpu.SemaphoreType.DMA] * NWIN,
        ],
    )
    def gather_k(gid_hbm, po_hbm, table_hbm, out_hbm,
                 gid_v, po_v, bufs, stage, sems):
        wid = lax.axis_index("s") * NC + lax.axis_index("c")
        base = wid * B_PER_W
        pltpu.sync_copy(gid_hbm.at[pl.ds(wid * NWIN, NWIN)], gid_v)
        pltpu.sync_copy(po_hbm.at[pl.ds(base, B_PER_W)], po_v)

        copies = [
            pltpu.async_copy(table_hbm.at[gid_v.at[w]], bufs.at[w], sems[w])
            for w in range(NWIN)
        ]
        for w in range(NWIN):
            copies[w].wait()

            def ext(blk, _, w=w):
                pov = po_v[pl.ds(w * W + blk * LANES, LANES)]
                for l in range(LANES):
                    po_j = pov[l]
                    for k in range(HALF // LANES):
                        stage[blk * LANES + l, pl.ds(k * LANES, LANES)] = (
                            bufs[w, blk * LANES + l,
                                 pl.ds(po_j + k * LANES, LANES)])
                return 0

            lax.fori_loop(0, W // LANES, ext, 0, unroll=False)
            out_base = pl.multiple_of(base + w * W, W)
            pltpu.sync_copy(stage, out_hbm.at[pl.ds(out_base, W)])

    return gather_k(gid2, po, table2)


def _tc_body(embp_ref, feat_ref, w1_ref, b1_ref, w2_ref, b2_ref,
             wc_ref, bc_ref, out_ref):
    dn = (((1,), (1,)), ((), ()))
    u = embp_ref[...]
    lo = lax.bitcast_convert_type(lax.shift_left(u, 16), jnp.float32)
    hi = lax.bitcast_convert_type(
        lax.bitwise_and(u, jnp.int32(-65536)), jnp.float32)
    emb = jnp.concatenate([lo, hi], axis=1)
    f = feat_ref[...]
    h = jnp.maximum(
        lax.dot_general(f, w1_ref[...], dn,
                        preferred_element_type=jnp.float32)
        + b1_ref[...][jnp.newaxis, :], 0.0)
    f2 = (lax.dot_general(h, w2_ref[...], dn,
                          preferred_element_type=jnp.float32)
          + b2_ref[...][jnp.newaxis, :])
    wc = wc_ref[...]
    o = (lax.dot_general(emb, wc[:, :EMBED_DIM], dn,
                         preferred_element_type=jnp.float32)
         + lax.dot_general(f2, wc[:, EMBED_DIM:], dn,
                           preferred_element_type=jnp.float32)
         + bc_ref[...][jnp.newaxis, :])
    s = jnp.sum(o * o, axis=1, keepdims=True)
    out_ref[...] = o * lax.rsqrt(jnp.maximum(s, 1e-24))


def _tc_dense(embp, feats, W1, b1, W2, b2, Wc, bc):
    tb = 2048
    grid = BATCH // tb
    full = lambda shape: pl.BlockSpec(shape, lambda i: tuple([0] * len(shape)))
    return pl.pallas_call(
        _tc_body,
        grid=(grid,),
        in_specs=[
            pl.BlockSpec((tb, HALF), lambda i: (i, 0)),
            pl.BlockSpec((tb, 4), lambda i: (i, 0)),
            full((32, 4)),
            full((32,)),
            full((EMBED_DIM, 32)),
            full((EMBED_DIM,)),
            full((EMBED_DIM, 2 * EMBED_DIM)),
            full((EMBED_DIM,)),
        ],
        out_specs=pl.BlockSpec((tb, EMBED_DIM), lambda i: (i, 0)),
        out_shape=jax.ShapeDtypeStruct((BATCH, EMBED_DIM), jnp.float32),
    )(embp, feats, W1, b1, W2, b2, Wc, bc)


def kernel(item_ids, item_features, emb_table, W1, b1, W2, b2, Wc, bc):
    ids = item_ids.astype(jnp.int32)
    q = ids // REPACK_C
    jj = ids - q * REPACK_C
    h = jj // REPACK_Q
    jr = jj - h * REPACK_Q
    gid2 = (q * REPACK_Q + jr).reshape(BATCH // W, W)
    po = h * HALF

    table2 = _tc_repack(emb_table.T)
    embp = _sc_gather(gid2, po, table2)
    return _tc_dense(embp, item_features, W1, b1, W2, b2, Wc, bc)


# speedup vs baseline: 1.1130x; 1.0069x over previous
import functools

import jax
import jax.numpy as jnp
from jax import lax
from jax.experimental import pallas as pl
from jax.experimental.pallas import tpu as pltpu
from jax.experimental.pallas import tpu_sc as plsc

N_ITEMS = 1000000
EMBED_DIM = 64
BATCH = 16384

NC = 2
NS = 16
NW = NC * NS
B_PER_W = BATCH // NW
LANES = 16
W = 128
NWIN = B_PER_W // W

REPACK_C = 31744
REPACK_Q = REPACK_C // 4
N_STEPS = -(-N_ITEMS // REPACK_C)
T2_ROWS = N_STEPS * REPACK_Q
HALF = EMBED_DIM // 2


def _repack_body(tt_ref, out_ref):
    x = tt_ref[...]
    u = lax.bitcast_convert_type(x, jnp.int32)
    lo = lax.shift_right_logical(u[:HALF, :], 16)
    hi = u[HALF:, :]
    w = lax.bitwise_or(lo, lax.bitwise_and(hi, jnp.int32(-65536)))
    for q in range(4):
        out_ref[:, q * HALF:(q + 1) * HALF] = (
            w[:, q * REPACK_Q:(q + 1) * REPACK_Q].T)


def _tc_repack(table_t):
    return pl.pallas_call(
        _repack_body,
        grid=(N_STEPS,),
        in_specs=[pl.BlockSpec((EMBED_DIM, REPACK_C), lambda i: (0, i))],
        out_specs=pl.BlockSpec((REPACK_Q, 128), lambda i: (i, 0)),
        out_shape=jax.ShapeDtypeStruct((T2_ROWS, 128), jnp.int32),
    )(table_t)


def _sc_gather(gid2, po, table2):
    mesh = plsc.VectorSubcoreMesh(core_axis_name="c", subcore_axis_name="s")

    @functools.partial(
        pl.kernel,
        mesh=mesh,
        out_type=jax.ShapeDtypeStruct((BATCH, HALF), jnp.int32),
        scratch_types=[
            pltpu.VMEM((NWIN, W), jnp.int32),
            pltpu.VMEM((B_PER_W,), jnp.int32),
            pltpu.VMEM((NWIN, W, 128), jnp.int32),
            pltpu.VMEM((W, HALF), jnp.int32),
            [pltpu.SemaphoreType.DMA] * NWIN,
        ],
    )
    def gather_k(gid_hbm, po_hbm, table_hbm, out_hbm,
                 gid_v, po_v, bufs, stage, sems):
        wid = lax.axis_index("s") * NC + lax.axis_index("c")
        base = wid * B_PER_W
        pltpu.sync_copy(gid_hbm.at[pl.ds(wid * NWIN, NWIN)], gid_v)
        pltpu.sync_copy(po_hbm.at[pl.ds(base, B_PER_W)], po_v)

        copies = [
            pltpu.async_copy(table_hbm.at[gid_v.at[w]], bufs.at[w], sems[w])
            for w in range(NWIN)
        ]
        for w in range(NWIN):
            copies[w].wait()

            def ext(blk, _, w=w):
                pov = po_v[pl.ds(w * W + blk * LANES, LANES)]
                for l in range(LANES):
                    po_j = pov[l]
                    for k in range(HALF // LANES):
                        stage[blk * LANES + l, pl.ds(k * LANES, LANES)] = (
                            bufs[w, blk * LANES + l,
                                 pl.ds(po_j + k * LANES, LANES)])
                return 0

            lax.fori_loop(0, W // LANES, ext, 0, unroll=False)
            out_base = pl.multiple_of(base + w * W, W)
            pltpu.sync_copy(stage, out_hbm.at[pl.ds(out_base, W)])

    return gather_k(gid2, po, table2)


def _tc_body(embp_ref, feat_ref, w1_ref, b1_ref, w2_ref, b2_ref,
             wc_ref, bc_ref, out_ref):
    dn = (((1,), (1,)), ((), ()))
    u = embp_ref[...]
    lo = lax.bitcast_convert_type(lax.shift_left(u, 16), jnp.float32)
    hi = lax.bitcast_convert_type(
        lax.bitwise_and(u, jnp.int32(-65536)), jnp.float32)
    emb = jnp.concatenate([lo, hi], axis=1)
    f = feat_ref[...]
    h = jnp.maximum(
        lax.dot_general(f, w1_ref[...], dn,
                        preferred_element_type=jnp.float32)
        + b1_ref[...][jnp.newaxis, :], 0.0)
    f2 = (lax.dot_general(h, w2_ref[...], dn,
                          preferred_element_type=jnp.float32)
          + b2_ref[...][jnp.newaxis, :])
    wc = wc_ref[...]
    o = (lax.dot_general(emb, wc[:, :EMBED_DIM], dn,
                         preferred_element_type=jnp.float32)
         + lax.dot_general(f2, wc[:, EMBED_DIM:], dn,
                           preferred_element_type=jnp.float32)
         + bc_ref[...][jnp.newaxis, :])
    s = jnp.sum(o * o, axis=1, keepdims=True)
    out_ref[...] = o * lax.rsqrt(jnp.maximum(s, 1e-24))


def _tc_dense(embp, feats, W1, b1, W2, b2, Wc, bc):
    tb = 2048
    grid = BATCH // tb
    full = lambda shape: pl.BlockSpec(shape, lambda i: tuple([0] * len(shape)))
    return pl.pallas_call(
        _tc_body,
        grid=(grid,),
        in_specs=[
            pl.BlockSpec((tb, HALF), lambda i: (i, 0)),
            pl.BlockSpec((tb, 4), lambda i: (i, 0)),
            full((32, 4)),
            full((32,)),
            full((EMBED_DIM, 32)),
            full((EMBED_DIM,)),
            full((EMBED_DIM, 2 * EMBED_DIM)),
            full((EMBED_DIM,)),
        ],
        out_specs=pl.BlockSpec((tb, EMBED_DIM), lambda i: (i, 0)),
        out_shape=jax.ShapeDtypeStruct((BATCH, EMBED_DIM), jnp.float32),
    )(embp, feats, W1, b1, W2, b2, Wc, bc)


def kernel(item_ids, item_features, emb_table, W1, b1, W2, b2, Wc, bc):
    ids = item_ids.astype(jnp.int32)
    q = ids // REPACK_C
    jj = ids - q * REPACK_C
    h = jj // REPACK_Q
    jr = jj - h * REPACK_Q
    gid2 = (q * REPACK_Q + jr).reshape(BATCH // W, W)
    po = h * HALF

    table2 = _tc_repack(emb_table.T)
    embp = _sc_gather(gid2, po, table2)
    return _tc_dense(embp, item_features, W1, b1, W2, b2, Wc, bc)
